# baseline (device time: 16158 ns/iter reference)
import jax
import jax.numpy as jnp
from jax import lax
from jax.experimental import pallas as pl
from jax.experimental.pallas import tpu as pltpu

X_DEV = 2
K = 4


def kernel(x):
    m_per, n = x.shape
    quarter = m_per // 4
    cs = quarter // K

    def body(
        x_ref, out_ref, recvx_buf, recvy_buf,
        own_sem, loc_sem,
        sendx_sems, recvx_sems,
        sendy_sems, recvy_sems,
        sendz_sems, recvz_sems,
        sendz2_sems, recvz2_sems,
    ):
        my_x = lax.axis_index("x")
        my_y = lax.axis_index("y")
        my_z = lax.axis_index("z")
        pz = lax.rem(my_z, 2)
        xpeer = (1 - my_x, my_y, my_z)
        ypeer = (my_x, 1 - my_y, my_z)
        zpeer = (my_x, my_y, my_z + 1 - 2 * pz)

        qi = 2 * my_y + pz
        qy = 2 * (1 - my_y) + pz
        mbase = (1 - my_x) * m_per

        own = pltpu.make_async_copy(
            x_ref, out_ref.at[pl.ds(my_x * m_per, m_per)], own_sem
        )
        own.start()

        barrier_sem = pltpu.get_barrier_semaphore()
        for nbr in (xpeer, ypeer, zpeer):
            pl.semaphore_signal(
                barrier_sem, inc=1, device_id=nbr,
                device_id_type=pl.DeviceIdType.MESH,
            )
        pl.semaphore_wait(barrier_sem, 3)

        xr = []
        for c in range(K):
            r = pltpu.make_async_remote_copy(
                src_ref=x_ref.at[pl.ds(qi * quarter + c * cs, cs)],
                dst_ref=recvx_buf.at[c],
                send_sem=sendx_sems.at[c],
                recv_sem=recvx_sems.at[c],
                device_id=xpeer,
                device_id_type=pl.DeviceIdType.MESH,
            )
            r.start()
            xr.append(r)

        yr, zr, locs = [], [], []
        for c in range(K):
            xr[c].wait_recv()
            rows = pl.ds(mbase + qi * quarter + c * cs, cs)
            f = pltpu.make_async_remote_copy(
                src_ref=recvx_buf.at[c],
                dst_ref=recvy_buf.at[c],
                send_sem=sendy_sems.at[c],
                recv_sem=recvy_sems.at[c],
                device_id=ypeer,
                device_id_type=pl.DeviceIdType.MESH,
            )
            f.start()
            yr.append(f)
            g = pltpu.make_async_remote_copy(
                src_ref=recvx_buf.at[c],
                dst_ref=out_ref.at[rows],
                send_sem=sendz_sems.at[c],
                recv_sem=recvz_sems.at[c],
                device_id=zpeer,
                device_id_type=pl.DeviceIdType.MESH,
            )
            g.start()
            zr.append(g)
            l = pltpu.make_async_copy(recvx_buf.at[c], out_ref.at[rows], loc_sem)
            l.start()
            locs.append(l)

        for c in range(K):
            yr[c].wait_recv()
            rows = pl.ds(mbase + qy * quarter + c * cs, cs)
            g = pltpu.make_async_remote_copy(
                src_ref=recvy_buf.at[c],
                dst_ref=out_ref.at[rows],
                send_sem=sendz2_sems.at[c],
                recv_sem=recvz2_sems.at[c],
                device_id=zpeer,
                device_id_type=pl.DeviceIdType.MESH,
            )
            g.start()
            zr.append(g)
            l = pltpu.make_async_copy(recvy_buf.at[c], out_ref.at[rows], loc_sem)
            l.start()
            locs.append(l)

        for c in range(K):
            xr[c].wait_send()
            yr[c].wait_send()
        for r in zr:
            r.wait()
        for l in locs:
            l.wait()
        own.wait()

    return pl.pallas_call(
        body,
        out_shape=jax.ShapeDtypeStruct((X_DEV * m_per, n), x.dtype),
        in_specs=[pl.BlockSpec(memory_space=pltpu.VMEM)],
        out_specs=pl.BlockSpec(memory_space=pltpu.MemorySpace.HBM),
        scratch_shapes=[
            pltpu.VMEM((K, cs, n), x.dtype),
            pltpu.VMEM((K, cs, n), x.dtype),
            pltpu.SemaphoreType.DMA,
            pltpu.SemaphoreType.DMA,
            pltpu.SemaphoreType.DMA((K,)),
            pltpu.SemaphoreType.DMA((K,)),
            pltpu.SemaphoreType.DMA((K,)),
            pltpu.SemaphoreType.DMA((K,)),
            pltpu.SemaphoreType.DMA((K,)),
            pltpu.SemaphoreType.DMA((K,)),
            pltpu.SemaphoreType.DMA((K,)),
            pltpu.SemaphoreType.DMA((K,)),
        ],
        compiler_params=pltpu.CompilerParams(collective_id=0),
    )(x)


# device time: 15712 ns/iter; 1.0284x vs baseline; 1.0284x over previous
import jax
from jax import lax
from jax.experimental import pallas as pl
from jax.experimental.pallas import tpu as pltpu

X_DEV = 2
K = 4


def kernel(x):
    m_per, n = x.shape
    half = m_per // 2
    cs = half // K

    def body(
        x_ref, out_ref, recv_buf,
        own_sem, loc_sem,
        sendx_sems, recvx_sems, sendy_sems, recvy_sems,
    ):
        my_x = lax.axis_index("x")
        my_y = lax.axis_index("y")
        my_z = lax.axis_index("z")
        xpeer = (1 - my_x, my_y, my_z)
        ypeer = (my_x, 1 - my_y, my_z)

        own = pltpu.make_async_copy(
            x_ref, out_ref.at[pl.ds(my_x * m_per, m_per)], own_sem
        )
        own.start()

        barrier_sem = pltpu.get_barrier_semaphore()
        for nbr in (xpeer, ypeer):
            pl.semaphore_signal(
                barrier_sem, inc=1, device_id=nbr,
                device_id_type=pl.DeviceIdType.MESH,
            )
        pl.semaphore_wait(barrier_sem, 2)

        xr = []
        for c in range(K):
            r = pltpu.make_async_remote_copy(
                src_ref=x_ref.at[pl.ds(my_y * half + c * cs, cs)],
                dst_ref=recv_buf.at[c],
                send_sem=sendx_sems.at[c],
                recv_sem=recvx_sems.at[c],
                device_id=xpeer,
                device_id_type=pl.DeviceIdType.MESH,
            )
            r.start()
            xr.append(r)

        yr = []
        locs = []
        for c in range(K):
            xr[c].wait_recv()
            dst_rows = pl.ds((1 - my_x) * m_per + my_y * half + c * cs, cs)
            f = pltpu.make_async_remote_copy(
                src_ref=recv_buf.at[c],
                dst_ref=out_ref.at[dst_rows],
                send_sem=sendy_sems.at[c],
                recv_sem=recvy_sems.at[c],
                device_id=ypeer,
                device_id_type=pl.DeviceIdType.MESH,
            )
            f.start()
            yr.append(f)
            l = pltpu.make_async_copy(recv_buf.at[c], out_ref.at[dst_rows], loc_sem)
            l.start()
            locs.append(l)

        for c in range(K):
            xr[c].wait_send()
            yr[c].wait()
            locs[c].wait()
        own.wait()

    return pl.pallas_call(
        body,
        out_shape=jax.ShapeDtypeStruct((X_DEV * m_per, n), x.dtype),
        in_specs=[pl.BlockSpec(memory_space=pltpu.VMEM)],
        out_specs=pl.BlockSpec(memory_space=pltpu.MemorySpace.HBM),
        scratch_shapes=[
            pltpu.VMEM((K, cs, n), x.dtype),
            pltpu.SemaphoreType.DMA,
            pltpu.SemaphoreType.DMA,
            pltpu.SemaphoreType.DMA((K,)),
            pltpu.SemaphoreType.DMA((K,)),
            pltpu.SemaphoreType.DMA((K,)),
            pltpu.SemaphoreType.DMA((K,)),
        ],
        compiler_params=pltpu.CompilerParams(collective_id=0),
    )(x)


# device time: 14902 ns/iter; 1.0843x vs baseline; 1.0544x over previous
import jax
from jax import lax
from jax.experimental import pallas as pl
from jax.experimental.pallas import tpu as pltpu

X_DEV = 2
K = 16


def kernel(x):
    m_per, n = x.shape
    half = m_per // 2
    cs = half // K

    def body(
        x_ref, out_ref, recv_buf,
        own_sem, loc_sem,
        sendx_sems, recvx_sems, sendy_sems, recvy_sems,
    ):
        my_x = lax.axis_index("x")
        my_y = lax.axis_index("y")
        my_z = lax.axis_index("z")
        xpeer = (1 - my_x, my_y, my_z)
        ypeer = (my_x, 1 - my_y, my_z)

        own = pltpu.make_async_copy(
            x_ref, out_ref.at[pl.ds(my_x * m_per, m_per)], own_sem
        )
        own.start()

        barrier_sem = pltpu.get_barrier_semaphore()
        for nbr in (xpeer, ypeer):
            pl.semaphore_signal(
                barrier_sem, inc=1, device_id=nbr,
                device_id_type=pl.DeviceIdType.MESH,
            )
        pl.semaphore_wait(barrier_sem, 2)

        xr = []
        for c in range(K):
            r = pltpu.make_async_remote_copy(
                src_ref=x_ref.at[pl.ds(my_y * half + c * cs, cs)],
                dst_ref=recv_buf.at[c],
                send_sem=sendx_sems.at[c],
                recv_sem=recvx_sems.at[c],
                device_id=xpeer,
                device_id_type=pl.DeviceIdType.MESH,
            )
            r.start()
            xr.append(r)

        yr = []
        locs = []
        for c in range(K):
            xr[c].wait_recv()
            dst_rows = pl.ds((1 - my_x) * m_per + my_y * half + c * cs, cs)
            f = pltpu.make_async_remote_copy(
                src_ref=recv_buf.at[c],
                dst_ref=out_ref.at[dst_rows],
                send_sem=sendy_sems.at[c],
                recv_sem=recvy_sems.at[c],
                device_id=ypeer,
                device_id_type=pl.DeviceIdType.MESH,
            )
            f.start()
            yr.append(f)
            l = pltpu.make_async_copy(recv_buf.at[c], out_ref.at[dst_rows], loc_sem)
            l.start()
            locs.append(l)

        for c in range(K):
            xr[c].wait_send()
            yr[c].wait()
            locs[c].wait()
        own.wait()

    return pl.pallas_call(
        body,
        out_shape=jax.ShapeDtypeStruct((X_DEV * m_per, n), x.dtype),
        in_specs=[pl.BlockSpec(memory_space=pltpu.VMEM)],
        out_specs=pl.BlockSpec(memory_space=pltpu.MemorySpace.HBM),
        scratch_shapes=[
            pltpu.VMEM((K, cs, n), x.dtype),
            pltpu.SemaphoreType.DMA,
            pltpu.SemaphoreType.DMA,
            pltpu.SemaphoreType.DMA((K,)),
            pltpu.SemaphoreType.DMA((K,)),
            pltpu.SemaphoreType.DMA((K,)),
            pltpu.SemaphoreType.DMA((K,)),
        ],
        compiler_params=pltpu.CompilerParams(collective_id=0),
    )(x)
